# baseline (speedup 1.0000x reference)
import jax
import jax.numpy as jnp
from jax.experimental import pallas as pl
from jax.experimental.pallas import tpu as pltpu

_C = 128
_L = 16
_U = 8
_K = 40


def _nn_loss_body(t8_ref, tx_ref, ty_ref, tz_ref, s8_ref, out_ref,
                  rhs_ref, mm_ref, acc_ref):
    i = pl.program_id(0)
    n_src = s8_ref.shape[1]
    T = _L * 8
    f32 = jnp.float32
    bf16 = jnp.bfloat16

    @pl.when(i == 0)
    def _init():
        s8 = s8_ref[...]
        sq = (s8[0:1, :] * s8[0:1, :]
              + s8[1:2, :] * s8[1:2, :]
              + s8[2:3, :] * s8[2:3, :])
        sh = s8.astype(bf16)
        sl = (s8 - sh.astype(f32)).astype(bf16)
        s2h = sq.astype(bf16)
        s2l = (sq - s2h.astype(f32)).astype(bf16)
        rhs_ref[0:8, :] = sh
        rhs_ref[8:16, :] = sl
        rhs_ref[16:24, :] = sh
        rhs_ref[24:32, :] = sl
        rhs_ref[32:33, :] = s2h
        rhs_ref[33:34, :] = s2l
        rhs_ref[34:40, :] = jnp.zeros((6, n_src), bf16)
        acc_ref[...] = jnp.zeros((_L, 8, 1), f32)

    t8 = t8_ref[...]
    th = t8.astype(bf16)
    tl = (t8 - th.astype(f32)).astype(bf16)
    th2 = (-2.0 * th.astype(f32)).astype(bf16)
    tl2 = (-2.0 * tl.astype(f32)).astype(bf16)
    zz = jnp.zeros((T, 8), bf16)
    ones2 = jnp.ones((T, 2), bf16)
    z6 = jnp.zeros((T, 6), bf16)
    lhs = jnp.concatenate(
        [jnp.concatenate([th2, zz, zz, zz, ones2, z6], axis=1),
         jnp.concatenate([th2, th2, tl2, tl2, ones2, z6], axis=1)],
        axis=0)
    mm_ref[...] = jax.lax.dot_general(lhs, rhs_ref[...],
                                      (((1,), (0,)), ((), ())),
                                      preferred_element_type=f32)

    def body(k, carry):
        bscore, bval = carry
        sl_ = pl.ds(k * _C, _C)
        score = mm_ref[0:T, sl_].reshape(_L, 8, _C)
        val = mm_ref[T:2 * T, sl_].reshape(_L, 8, _C)
        upd = score < bscore
        return jnp.minimum(bscore, score), jnp.where(upd, val, bval)

    inf0 = jnp.full((_L, 8, _C), jnp.inf, dtype=f32)
    bscore, bval = jax.lax.fori_loop(0, n_src // _C, body, (inf0, inf0),
                                     unroll=_U)
    ms = jnp.min(bscore, axis=2, keepdims=True)
    vsel = jnp.where(bscore == ms, bval, jnp.inf)
    mval = jnp.min(vsel, axis=2, keepdims=True)
    tx = tx_ref[...]
    ty = ty_ref[...]
    tz = tz_ref[...]
    tsq = tx * tx + ty * ty + tz * tz
    acc_ref[...] += 0.5 * (tsq + mval)

    @pl.when(i == pl.num_programs(0) - 1)
    def _fin():
        out_ref[...] = acc_ref[...]


def kernel(src_V, tar_V):
    n_src = src_V.shape[0]
    n_tar = tar_V.shape[0]
    f32 = jnp.float32
    t8 = jnp.concatenate([tar_V, jnp.zeros((n_tar, 5), f32)], axis=1)
    s8t = jnp.concatenate([src_V, jnp.zeros((n_src, 5), f32)], axis=1).T
    tx = tar_V[:, 0].reshape(n_tar // 8, 8, 1)
    ty = tar_V[:, 1].reshape(n_tar // 8, 8, 1)
    tz = tar_V[:, 2].reshape(n_tar // 8, 8, 1)

    grid = n_tar // (8 * _L)
    T = 8 * _L
    t8_spec = pl.BlockSpec((T, 8), lambda i: (i, 0))
    tar_spec = pl.BlockSpec((_L, 8, 1), lambda i: (i, 0, 0))
    s8_spec = pl.BlockSpec((8, n_src), lambda i: (0, 0))

    out = pl.pallas_call(
        _nn_loss_body,
        grid=(grid,),
        in_specs=[t8_spec, tar_spec, tar_spec, tar_spec, s8_spec],
        out_specs=pl.BlockSpec((_L, 8, 1), lambda i: (0, 0, 0)),
        out_shape=jax.ShapeDtypeStruct((_L, 8, 1), jnp.float32),
        scratch_shapes=[pltpu.VMEM((_K, n_src), jnp.bfloat16),
                        pltpu.VMEM((2 * T, n_src), jnp.float32),
                        pltpu.VMEM((_L, 8, 1), jnp.float32)],
    )(t8, tx, ty, tz, s8t)
    return jnp.sum(out)


# speedup vs baseline: 1.9111x; 1.9111x over previous
import jax
import jax.numpy as jnp
from jax.experimental import pallas as pl
from jax.experimental.pallas import tpu as pltpu

_C = 256
_L = 8
_Q = 4
_K = 40


def _nn_loss_body(t8_ref, tx_ref, ty_ref, tz_ref, s8_ref, out_ref,
                  rhs_ref, acc_ref):
    i = pl.program_id(0)
    n_src = s8_ref.shape[1]
    T = _L * 8
    f32 = jnp.float32
    bf16 = jnp.bfloat16

    @pl.when(i == 0)
    def _init():
        s8 = s8_ref[...]
        sq = (s8[0:1, :] * s8[0:1, :]
              + s8[1:2, :] * s8[1:2, :]
              + s8[2:3, :] * s8[2:3, :])
        sh = s8.astype(bf16)
        sl = (s8 - sh.astype(f32)).astype(bf16)
        s2h = sq.astype(bf16)
        s2l = (sq - s2h.astype(f32)).astype(bf16)
        rhs_ref[0:8, :] = sh
        rhs_ref[8:16, :] = sl
        rhs_ref[16:24, :] = sh
        rhs_ref[24:32, :] = sl
        rhs_ref[32:33, :] = s2h
        rhs_ref[33:34, :] = s2l
        rhs_ref[34:40, :] = jnp.zeros((6, n_src), bf16)
        acc_ref[...] = jnp.zeros((_L, 8, 1), f32)

    t8 = t8_ref[...]
    th = t8.astype(bf16)
    tl = (t8 - th.astype(f32)).astype(bf16)
    th2 = (-2.0 * th.astype(f32)).astype(bf16)
    tl2 = (-2.0 * tl.astype(f32)).astype(bf16)
    zz = jnp.zeros((T, 8), bf16)
    ones2 = jnp.ones((T, 2), bf16)
    z6 = jnp.zeros((T, 6), bf16)
    lhs = jnp.concatenate(
        [jnp.concatenate([th2, zz, zz, zz, ones2, z6], axis=1),
         jnp.concatenate([th2, th2, tl2, tl2, ones2, z6], axis=1)],
        axis=0)

    QS = n_src // _Q

    def quarter(q):
        return jax.lax.dot_general(lhs, rhs_ref[:, q * QS:(q + 1) * QS],
                                   (((1,), (0,)), ((), ())),
                                   preferred_element_type=f32)

    bscore = jnp.full((_L, 8, _C), jnp.inf, dtype=f32)
    bval = bscore
    mm_prev = quarter(0)
    for q in range(_Q):
        mm_cur = quarter(q + 1) if q + 1 < _Q else None
        for c in range(QS // _C):
            score = mm_prev[0:T, c * _C:(c + 1) * _C].reshape(_L, 8, _C)
            val = mm_prev[T:2 * T, c * _C:(c + 1) * _C].reshape(_L, 8, _C)
            upd = score < bscore
            bscore = jnp.minimum(bscore, score)
            bval = jnp.where(upd, val, bval)
        mm_prev = mm_cur

    ms = jnp.min(bscore, axis=2, keepdims=True)
    vsel = jnp.where(bscore == ms, bval, jnp.inf)
    mval = jnp.min(vsel, axis=2, keepdims=True)
    tx = tx_ref[...]
    ty = ty_ref[...]
    tz = tz_ref[...]
    tsq = tx * tx + ty * ty + tz * tz
    acc_ref[...] += 0.5 * (tsq + mval)

    @pl.when(i == pl.num_programs(0) - 1)
    def _fin():
        out_ref[...] = acc_ref[...]


def kernel(src_V, tar_V):
    n_src = src_V.shape[0]
    n_tar = tar_V.shape[0]
    f32 = jnp.float32
    t8 = jnp.concatenate([tar_V, jnp.zeros((n_tar, 5), f32)], axis=1)
    s8t = jnp.concatenate([src_V, jnp.zeros((n_src, 5), f32)], axis=1).T
    tx = tar_V[:, 0].reshape(n_tar // 8, 8, 1)
    ty = tar_V[:, 1].reshape(n_tar // 8, 8, 1)
    tz = tar_V[:, 2].reshape(n_tar // 8, 8, 1)

    grid = n_tar // (8 * _L)
    T = 8 * _L
    t8_spec = pl.BlockSpec((T, 8), lambda i: (i, 0))
    tar_spec = pl.BlockSpec((_L, 8, 1), lambda i: (i, 0, 0))
    s8_spec = pl.BlockSpec((8, n_src), lambda i: (0, 0))

    out = pl.pallas_call(
        _nn_loss_body,
        grid=(grid,),
        in_specs=[t8_spec, tar_spec, tar_spec, tar_spec, s8_spec],
        out_specs=pl.BlockSpec((_L, 8, 1), lambda i: (0, 0, 0)),
        out_shape=jax.ShapeDtypeStruct((_L, 8, 1), jnp.float32),
        scratch_shapes=[pltpu.VMEM((_K, n_src), jnp.bfloat16),
                        pltpu.VMEM((_L, 8, 1), jnp.float32)],
    )(t8, tx, ty, tz, s8t)
    return jnp.sum(out)


# speedup vs baseline: 2.3664x; 1.2383x over previous
import jax
import jax.numpy as jnp
from jax.experimental import pallas as pl
from jax.experimental.pallas import tpu as pltpu

_C = 256
_L = 64
_Q = 8
_K = 40


def _nn_loss_body(t8_ref, tx_ref, ty_ref, tz_ref, s8_ref, out_ref,
                  rhs_ref, acc_ref):
    i = pl.program_id(0)
    n_src = s8_ref.shape[1]
    T = _L * 8
    f32 = jnp.float32
    bf16 = jnp.bfloat16

    @pl.when(i == 0)
    def _init():
        s8 = s8_ref[...]
        sq = (s8[0:1, :] * s8[0:1, :]
              + s8[1:2, :] * s8[1:2, :]
              + s8[2:3, :] * s8[2:3, :])
        sh = s8.astype(bf16)
        sl = (s8 - sh.astype(f32)).astype(bf16)
        s2h = sq.astype(bf16)
        s2l = (sq - s2h.astype(f32)).astype(bf16)
        rhs_ref[0:8, :] = sh
        rhs_ref[8:16, :] = sl
        rhs_ref[16:24, :] = sh
        rhs_ref[24:32, :] = sl
        rhs_ref[32:33, :] = s2h
        rhs_ref[33:34, :] = s2l
        rhs_ref[34:40, :] = jnp.zeros((6, n_src), bf16)
        acc_ref[...] = jnp.zeros((_L, 8, 1), f32)

    t8 = t8_ref[...]
    th = t8.astype(bf16)
    tl = (t8 - th.astype(f32)).astype(bf16)
    th2 = (-2.0 * th.astype(f32)).astype(bf16)
    tl2 = (-2.0 * tl.astype(f32)).astype(bf16)
    zz = jnp.zeros((T, 8), bf16)
    ones2 = jnp.ones((T, 2), bf16)
    z6 = jnp.zeros((T, 6), bf16)
    lhs = jnp.concatenate(
        [jnp.concatenate([th2, zz, zz, zz, ones2, z6], axis=1),
         jnp.concatenate([th2, th2, tl2, tl2, ones2, z6], axis=1)],
        axis=0)

    QS = n_src // _Q

    def quarter(q):
        return jax.lax.dot_general(lhs, rhs_ref[:, q * QS:(q + 1) * QS],
                                   (((1,), (0,)), ((), ())),
                                   preferred_element_type=f32)

    bscore = jnp.full((_L, 8, _C), jnp.inf, dtype=f32)
    bval = bscore
    mm_prev = quarter(0)
    for q in range(_Q):
        mm_cur = quarter(q + 1) if q + 1 < _Q else None
        for c in range(QS // _C):
            score = mm_prev[0:T, c * _C:(c + 1) * _C].reshape(_L, 8, _C)
            val = mm_prev[T:2 * T, c * _C:(c + 1) * _C].reshape(_L, 8, _C)
            upd = score < bscore
            bscore = jnp.minimum(bscore, score)
            bval = jnp.where(upd, val, bval)
        mm_prev = mm_cur

    ms = jnp.min(bscore, axis=2, keepdims=True)
    vsel = jnp.where(bscore == ms, bval, jnp.inf)
    mval = jnp.min(vsel, axis=2, keepdims=True)
    tx = tx_ref[...]
    ty = ty_ref[...]
    tz = tz_ref[...]
    tsq = tx * tx + ty * ty + tz * tz
    acc_ref[...] += 0.5 * (tsq + mval)

    @pl.when(i == pl.num_programs(0) - 1)
    def _fin():
        out_ref[...] = acc_ref[...]


def kernel(src_V, tar_V):
    n_src = src_V.shape[0]
    n_tar = tar_V.shape[0]
    f32 = jnp.float32
    t8 = jnp.concatenate([tar_V, jnp.zeros((n_tar, 5), f32)], axis=1)
    s8t = jnp.concatenate([src_V, jnp.zeros((n_src, 5), f32)], axis=1).T
    tx = tar_V[:, 0].reshape(n_tar // 8, 8, 1)
    ty = tar_V[:, 1].reshape(n_tar // 8, 8, 1)
    tz = tar_V[:, 2].reshape(n_tar // 8, 8, 1)

    grid = n_tar // (8 * _L)
    T = 8 * _L
    t8_spec = pl.BlockSpec((T, 8), lambda i: (i, 0))
    tar_spec = pl.BlockSpec((_L, 8, 1), lambda i: (i, 0, 0))
    s8_spec = pl.BlockSpec((8, n_src), lambda i: (0, 0))

    out = pl.pallas_call(
        _nn_loss_body,
        grid=(grid,),
        in_specs=[t8_spec, tar_spec, tar_spec, tar_spec, s8_spec],
        out_specs=pl.BlockSpec((_L, 8, 1), lambda i: (0, 0, 0)),
        out_shape=jax.ShapeDtypeStruct((_L, 8, 1), jnp.float32),
        scratch_shapes=[pltpu.VMEM((_K, n_src), jnp.bfloat16),
                        pltpu.VMEM((_L, 8, 1), jnp.float32)],
    )(t8, tx, ty, tz, s8t)
    return jnp.sum(out)


# speedup vs baseline: 2.3885x; 1.0093x over previous
import jax
import jax.numpy as jnp
from jax.experimental import pallas as pl
from jax.experimental.pallas import tpu as pltpu

_C = 256
_L = 128
_Q = 8
_K = 40


def _nn_loss_body(t8_ref, tx_ref, ty_ref, tz_ref, s8_ref, out_ref,
                  rhs_ref, acc_ref):
    i = pl.program_id(0)
    n_src = s8_ref.shape[1]
    T = _L * 8
    f32 = jnp.float32
    bf16 = jnp.bfloat16

    @pl.when(i == 0)
    def _init():
        s8 = s8_ref[...]
        sq = (s8[0:1, :] * s8[0:1, :]
              + s8[1:2, :] * s8[1:2, :]
              + s8[2:3, :] * s8[2:3, :])
        sh = s8.astype(bf16)
        sl = (s8 - sh.astype(f32)).astype(bf16)
        s2h = sq.astype(bf16)
        s2l = (sq - s2h.astype(f32)).astype(bf16)
        rhs_ref[0:8, :] = sh
        rhs_ref[8:16, :] = sl
        rhs_ref[16:24, :] = sh
        rhs_ref[24:32, :] = sl
        rhs_ref[32:33, :] = s2h
        rhs_ref[33:34, :] = s2l
        rhs_ref[34:40, :] = jnp.zeros((6, n_src), bf16)
        acc_ref[...] = jnp.zeros((_L, 8, 1), f32)

    t8 = t8_ref[...]
    th = t8.astype(bf16)
    tl = (t8 - th.astype(f32)).astype(bf16)
    th2 = (-2.0 * th.astype(f32)).astype(bf16)
    tl2 = (-2.0 * tl.astype(f32)).astype(bf16)
    zz = jnp.zeros((T, 8), bf16)
    ones2 = jnp.ones((T, 2), bf16)
    z6 = jnp.zeros((T, 6), bf16)
    lhs = jnp.concatenate(
        [jnp.concatenate([th2, zz, zz, zz, ones2, z6], axis=1),
         jnp.concatenate([th2, th2, tl2, tl2, ones2, z6], axis=1)],
        axis=0)

    QS = n_src // _Q

    def quarter(q):
        return jax.lax.dot_general(lhs, rhs_ref[:, q * QS:(q + 1) * QS],
                                   (((1,), (0,)), ((), ())),
                                   preferred_element_type=f32)

    bscore = jnp.full((_L, 8, _C), jnp.inf, dtype=f32)
    bval = bscore
    mm_prev = quarter(0)
    for q in range(_Q):
        mm_cur = quarter(q + 1) if q + 1 < _Q else None
        for c in range(QS // _C):
            score = mm_prev[0:T, c * _C:(c + 1) * _C].reshape(_L, 8, _C)
            val = mm_prev[T:2 * T, c * _C:(c + 1) * _C].reshape(_L, 8, _C)
            upd = score < bscore
            bscore = jnp.minimum(bscore, score)
            bval = jnp.where(upd, val, bval)
        mm_prev = mm_cur

    ms = jnp.min(bscore, axis=2, keepdims=True)
    vsel = jnp.where(bscore == ms, bval, jnp.inf)
    mval = jnp.min(vsel, axis=2, keepdims=True)
    tx = tx_ref[...]
    ty = ty_ref[...]
    tz = tz_ref[...]
    tsq = tx * tx + ty * ty + tz * tz
    acc_ref[...] += 0.5 * (tsq + mval)

    @pl.when(i == pl.num_programs(0) - 1)
    def _fin():
        out_ref[...] = acc_ref[...]


def kernel(src_V, tar_V):
    n_src = src_V.shape[0]
    n_tar = tar_V.shape[0]
    f32 = jnp.float32
    t8 = jnp.concatenate([tar_V, jnp.zeros((n_tar, 5), f32)], axis=1)
    s8t = jnp.concatenate([src_V, jnp.zeros((n_src, 5), f32)], axis=1).T
    tx = tar_V[:, 0].reshape(n_tar // 8, 8, 1)
    ty = tar_V[:, 1].reshape(n_tar // 8, 8, 1)
    tz = tar_V[:, 2].reshape(n_tar // 8, 8, 1)

    grid = n_tar // (8 * _L)
    T = 8 * _L
    t8_spec = pl.BlockSpec((T, 8), lambda i: (i, 0))
    tar_spec = pl.BlockSpec((_L, 8, 1), lambda i: (i, 0, 0))
    s8_spec = pl.BlockSpec((8, n_src), lambda i: (0, 0))

    out = pl.pallas_call(
        _nn_loss_body,
        grid=(grid,),
        in_specs=[t8_spec, tar_spec, tar_spec, tar_spec, s8_spec],
        out_specs=pl.BlockSpec((_L, 8, 1), lambda i: (0, 0, 0)),
        out_shape=jax.ShapeDtypeStruct((_L, 8, 1), jnp.float32),
        scratch_shapes=[pltpu.VMEM((_K, n_src), jnp.bfloat16),
                        pltpu.VMEM((_L, 8, 1), jnp.float32)],
    )(t8, tx, ty, tz, s8t)
    return jnp.sum(out)
